# baseline (speedup 1.0000x reference)
import functools

import jax
import jax.numpy as jnp
from jax import lax
from jax.experimental import pallas as pl
from jax.experimental.pallas import tpu as pltpu
from jax.experimental.pallas import tpu_sc as plsc

_B = 16384
_D = 32
_GAMMA = 1.0
_NC = 2
_NS = 16
_NW = _NC * _NS
_C = _B // _NW
_E = 1000000
_R = 1000
_SB = 1024
_NSB_FULL = _E // _SB
_TAIL = _E - _NSB_FULL * _SB
_NSB = _NSB_FULL + 1
_MAXSB = 31
_CAP = 8192
_NSLOT = 4 * _B
_DUMMY = _NSLOT
_SROWS = _NSLOT + 16
_CHUNK = 16
_G = 64
_NG = _C // _G
_NK = _G // _CHUNK


def _rsqrt_nr(x):
    xh = x * jnp.float32(0.5)
    i = plsc.bitcast(x, jnp.int32)
    i = jnp.int32(0x5F3759DF) - jnp.right_shift(i, jnp.int32(1))
    y = plsc.bitcast(i, jnp.float32)
    for _ in range(3):
        y = y * (jnp.float32(1.5) - xh * y * y)
    return y


def _pcnt(mask):
    r = plsc.all_reduce_population_count(mask)
    return r[0] if getattr(r, "ndim", 0) else r


_IOTA = None


def _harvest_body(head, tail, n_head, n_tail, ent4, etail, stage,
                  ibuf, hidx, hslot, sbidx, sbslot, slab, tslab, srcb,
                  sslot, sem, sem2):
    wid = lax.axis_index("s") * _NC + lax.axis_index("c")
    iota = lax.iota(jnp.int32, _CHUNK)
    srcs = (head, tail, n_head, n_tail)

    nsb_w = jnp.where(wid <= (_NSB_FULL - (_MAXSB - 1) * _NW),
                      jnp.int32(_MAXSB), jnp.int32(_MAXSB - 1))

    def pass_body(carry):
        p, _last = carry
        lo = p * jnp.int32(_CAP)
        hi = lo + jnp.int32(_CAP)

        def scan_tensor(t, state):
            pltpu.sync_copy(srcs[t], ibuf)

            def chunk(c, state3):
                hcnt, cntm = state3
                v = ibuf[pl.ds(c * _CHUNK, _CHUNK)]
                sbv = jnp.right_shift(v, jnp.int32(10))
                mine = jnp.bitwise_and(sbv, jnp.int32(31)) == wid
                m32 = mine.astype(jnp.int32)
                rank = cntm + plsc.cumsum(m32) - m32
                keep = mine & (rank >= lo) & (rank < hi)
                plsc.store_compressed(hidx.at[pl.ds(hcnt, _CHUNK)], v, mask=keep)
                slotv = jnp.int32(t * _B) + c * jnp.int32(_CHUNK) + iota
                plsc.store_compressed(
                    hslot.at[pl.ds(hcnt, _CHUNK)], slotv, mask=keep)
                return hcnt + _pcnt(keep), cntm + _pcnt(mine)

            return lax.fori_loop(0, _B // _CHUNK, chunk, state, unroll=8)

        hcnt = jnp.int32(0)
        cntm = jnp.int32(0)
        for t in range(4):
            hcnt, cntm = scan_tensor(t, (hcnt, cntm))
        for z in range(4):
            hidx[pl.ds(hcnt + z * _CHUNK, _CHUNK)] = jnp.full(
                (_CHUNK,), -1, jnp.int32)
            hslot[pl.ds(hcnt + z * _CHUNK, _CHUNK)] = jnp.full(
                (_CHUNK,), _DUMMY, jnp.int32)

        def sb_body(ms, carry2):
            @pl.when(ms < nsb_w)
            def _():
                sb = wid + ms * jnp.int32(_NW)

                @pl.when(sb < jnp.int32(_NSB_FULL))
                def _():
                    hs = [
                        pltpu.async_copy(
                            ent4.at[db, :, pl.ds(sb * _SB, _SB)],
                            slab.at[db], sem)
                        for db in range(4)
                    ]
                    for h in hs:
                        h.wait()

                @pl.when(sb == jnp.int32(_NSB_FULL))
                def _():
                    hs = [
                        pltpu.async_copy(
                            ent4.at[db, :, pl.ds(_NSB_FULL * _SB, 512)],
                            slab.at[db, :, pl.ds(0, 512)], sem)
                        for db in range(4)
                    ]
                    for h in hs:
                        h.wait()

                nch4 = jnp.right_shift(hcnt + jnp.int32(63), jnp.int32(6))

                def sel(cc, scnt):
                    for z in range(4):
                        off = (cc * 4 + z) * _CHUNK
                        hv = hidx[pl.ds(off, _CHUNK)]
                        sv = hslot[pl.ds(off, _CHUNK)]
                        m = ((jnp.right_shift(hv, jnp.int32(10)) == sb)
                             & (hv < jnp.int32(_E - 64)))
                        plsc.store_compressed(
                            sbidx.at[pl.ds(scnt, _CHUNK)], hv, mask=m)
                        plsc.store_compressed(
                            sbslot.at[pl.ds(scnt, _CHUNK)], sv, mask=m)
                        scnt = scnt + _pcnt(m)
                    return scnt

                scnt = lax.fori_loop(0, nch4, sel, jnp.int32(0))
                sbidx[pl.ds(scnt, _CHUNK)] = jnp.full(
                    (_CHUNK,), 0, jnp.int32)
                sbslot[pl.ds(scnt, _CHUNK)] = jnp.full(
                    (_CHUNK,), _DUMMY, jnp.int32)

                for z in range(1, 8):
                    sbidx[pl.ds(scnt + z * _CHUNK, _CHUNK)] = jnp.full(
                        (_CHUNK,), 0, jnp.int32)
                    sbslot[pl.ds(scnt + z * _CHUNK, _CHUNK)] = jnp.full(
                        (_CHUNK,), _DUMMY, jnp.int32)
                nmg = jnp.right_shift(scnt + jnp.int32(127), jnp.int32(7))

                def grp(mg, carry3):
                    for sub in range(8):
                        off = mg * 128 + sub * _CHUNK
                        lv = jnp.bitwise_and(
                            sbidx[pl.ds(off, _CHUNK)], jnp.int32(1023))
                        for s0 in range(_D):
                            e = (iota + jnp.int32(s0)) & jnp.int32(_D - 1)
                            dbc = jnp.right_shift(e, jnp.int32(3))
                            sc_ = jnp.bitwise_and(e, jnp.int32(7))
                            val = plsc.load_gather(slab, [dbc, sc_, lv])
                            plsc.store_scatter(
                                srcb, [iota + sub * _CHUNK, e], val)
                        sslot[pl.ds(sub * _CHUNK, _CHUNK)] = (
                            sbslot[pl.ds(off, _CHUNK)])
                    pltpu.async_copy(srcb, stage.at[sslot], sem2).wait()
                    return carry3

                lax.fori_loop(0, nmg, grp, 0)

            return carry2

        lax.fori_loop(0, _MAXSB, sb_body, 0)

        @pl.when(wid == jnp.int32(_NSB_FULL % _NW))
        def _():
            pltpu.sync_copy(etail, tslab)
            nch = jnp.right_shift(hcnt + jnp.int32(15), jnp.int32(4))

            def tsel(cc, scnt):
                hv = hidx[pl.ds(cc * _CHUNK, _CHUNK)]
                sv = hslot[pl.ds(cc * _CHUNK, _CHUNK)]
                m = hv >= jnp.int32(_E - 64)
                plsc.store_compressed(
                    sbidx.at[pl.ds(scnt, _CHUNK)], hv, mask=m)
                plsc.store_compressed(
                    sbslot.at[pl.ds(scnt, _CHUNK)], sv, mask=m)
                return scnt + _pcnt(m)

            scnt = lax.fori_loop(0, nch, tsel, jnp.int32(0))
            for z in range(8):
                sbidx[pl.ds(scnt + z * _CHUNK, _CHUNK)] = jnp.full(
                    (_CHUNK,), _E - 64, jnp.int32)
                sbslot[pl.ds(scnt + z * _CHUNK, _CHUNK)] = jnp.full(
                    (_CHUNK,), _DUMMY, jnp.int32)
            nmg = jnp.right_shift(scnt + jnp.int32(127), jnp.int32(7))

            def tgrp(mg, carry3):
                for sub in range(8):
                    off = mg * 128 + sub * _CHUNK
                    lv = sbidx[pl.ds(off, _CHUNK)] - jnp.int32(_E - 64)
                    for s0 in range(_D):
                        e = (iota + jnp.int32(s0)) & jnp.int32(_D - 1)
                        dbc = jnp.right_shift(e, jnp.int32(3))
                        sc_ = jnp.bitwise_and(e, jnp.int32(7))
                        val = plsc.load_gather(tslab, [dbc, sc_, lv])
                        plsc.store_scatter(
                            srcb, [iota + sub * _CHUNK, e], val)
                    sslot[pl.ds(sub * _CHUNK, _CHUNK)] = (
                        sbslot[pl.ds(off, _CHUNK)])
                pltpu.async_copy(srcb, stage.at[sslot], sem2).wait()
                return carry3

            lax.fori_loop(0, nmg, tgrp, 0)

        return p + jnp.int32(1), hcnt

    lax.while_loop(lambda c: c[1] == jnp.int32(_CAP), pass_body,
                   (jnp.int32(0), jnp.int32(_CAP)))


_harvest = functools.partial(
    pl.kernel,
    mesh=plsc.VectorSubcoreMesh(core_axis_name="c", subcore_axis_name="s"),
    compiler_params=pltpu.CompilerParams(needs_layout_passes=False),
    out_type=jax.ShapeDtypeStruct((_SROWS, 128), jnp.float32),
    scratch_types=[
        pltpu.VMEM((_B,), jnp.int32),
        pltpu.VMEM((_CAP + 64,), jnp.int32),
        pltpu.VMEM((_CAP + 64,), jnp.int32),
        pltpu.VMEM((_CAP + 192,), jnp.int32),
        pltpu.VMEM((_CAP + 192,), jnp.int32),
        pltpu.VMEM((4, 8, _SB), jnp.float32),
        pltpu.VMEM((4, 8, 64), jnp.float32),
        pltpu.VMEM((128, 128), jnp.float32),
        pltpu.VMEM((128,), jnp.int32),
        pltpu.SemaphoreType.DMA,
        pltpu.SemaphoreType.DMA,
    ],
)(_harvest_body)


def _score_body(rel, n_rel, stage, rel4, out,
                rx0, rx1, g0, g1, g2, g3, rbuf, outv, sem):
    wid = lax.axis_index("s") * _NC + lax.axis_index("c")
    base = wid * _C
    iota = lax.iota(jnp.int32, _CHUNK)
    zero = jnp.zeros((_CHUNK,), jnp.float32)

    for db in range(4):
        pltpu.sync_copy(rel4.at[db], rbuf.at[db])

    ent_slabs = (g0, g1, g2, g3)

    def group_body(j, carry):
        gbase = base + j * _G
        pltpu.sync_copy(rel.at[pl.ds(gbase, _G)], rx0)
        pltpu.sync_copy(n_rel.at[pl.ds(gbase, _G)], rx1)
        handles = [
            pltpu.async_copy(
                stage.at[pl.ds(t * _B + gbase, _G)], ent_slabs[t], sem)
            for t in range(4)
        ]
        for h in handles:
            h.wait()

        def chunk_body(k, carry3):
            rows = k * _CHUNK + iota
            sl = pl.ds(k * _CHUNK, _CHUNK)
            rv0 = rx0[sl]
            rv1 = rx1[sl]
            hh = tt = rr = hr = ht = rt = zero
            mhh = mtt = mrr = mhr = mht = mrt = zero
            for s0 in range(_D):
                e = (iota + jnp.int32(s0)) & jnp.int32(_D - 1)
                dbc = jnp.right_shift(e, jnp.int32(3))
                sc_ = jnp.bitwise_and(e, jnp.int32(7))
                h = plsc.load_gather(g0, [rows, e])
                t_ = plsc.load_gather(g1, [rows, e])
                nh = plsc.load_gather(g2, [rows, e])
                nt = plsc.load_gather(g3, [rows, e])
                r_ = plsc.load_gather(rbuf, [dbc, sc_, rv0])
                nr = plsc.load_gather(rbuf, [dbc, sc_, rv1])
                hh = hh + h * h
                tt = tt + t_ * t_
                rr = rr + r_ * r_
                hr = hr + h * r_
                ht = ht + h * t_
                rt = rt + r_ * t_
                mhh = mhh + nh * nh
                mtt = mtt + nt * nt
                mrr = mrr + nr * nr
                mhr = mhr + nh * nr
                mht = mht + nh * nt
                mrt = mrt + nr * nt
            two = jnp.float32(2.0)
            three = jnp.float32(3.0)
            eps = jnp.float32(1e-30)
            pos2 = three + two * (hr * _rsqrt_nr(hh * rr)
                                  - ht * _rsqrt_nr(hh * tt)
                                  - rt * _rsqrt_nr(rr * tt))
            neg2 = three + two * (mhr * _rsqrt_nr(mhh * mrr)
                                  - mht * _rsqrt_nr(mhh * mtt)
                                  - mrt * _rsqrt_nr(mrr * mtt))
            pos2 = jnp.maximum(pos2, eps)
            neg2 = jnp.maximum(neg2, eps)
            pos = pos2 * _rsqrt_nr(pos2)
            neg = neg2 * _rsqrt_nr(neg2)
            outv[pl.ds((j * _NK + k) * _CHUNK, _CHUNK)] = (
                jnp.float32(_GAMMA) + pos - neg)
            return carry3

        lax.fori_loop(0, _NK, chunk_body, 0)
        return carry

    lax.fori_loop(0, _NG, group_body, 0)
    pltpu.sync_copy(outv, out.at[pl.ds(base, _C)])


_score = functools.partial(
    pl.kernel,
    mesh=plsc.VectorSubcoreMesh(core_axis_name="c", subcore_axis_name="s"),
    compiler_params=pltpu.CompilerParams(needs_layout_passes=False),
    out_type=jax.ShapeDtypeStruct((_B,), jnp.float32),
    scratch_types=[
        pltpu.VMEM((_G,), jnp.int32),
        pltpu.VMEM((_G,), jnp.int32),
        pltpu.VMEM((_G, 128), jnp.float32),
        pltpu.VMEM((_G, 128), jnp.float32),
        pltpu.VMEM((_G, 128), jnp.float32),
        pltpu.VMEM((_G, 128), jnp.float32),
        pltpu.VMEM((4, 8, _R), jnp.float32),
        pltpu.VMEM((_C,), jnp.float32),
        pltpu.SemaphoreType.DMA,
    ],
)(_score_body)


def kernel(head, tail, relation, n_head, n_tail, n_relation, entity_embed, relation_embed):
    ent4 = entity_embed.T.reshape(4, 8, _E)
    rel4 = relation_embed.T.reshape(4, 8, _R)
    etail = entity_embed[_E - 64:].T.reshape(4, 8, 64)
    stage = _harvest(
        head.astype(jnp.int32),
        tail.astype(jnp.int32),
        n_head.astype(jnp.int32),
        n_tail.astype(jnp.int32),
        ent4,
        etail,
    )
    return _score(
        relation.astype(jnp.int32),
        n_relation.astype(jnp.int32),
        stage,
        rel4,
    )


# speedup vs baseline: 4.6966x; 4.6966x over previous
import functools

import jax
import jax.numpy as jnp
from jax import lax
from jax.experimental import pallas as pl
from jax.experimental.pallas import tpu as pltpu
from jax.experimental.pallas import tpu_sc as plsc

_B = 16384
_D = 32
_GAMMA = 1.0
_NC = 2
_NS = 16
_NW = _NC * _NS
_C = _B // _NW
_G = 64
_NG = _C // _G
_CHUNK = 16
_NK = _G // _CHUNK
_E4 = 1000000 // 4
_R4 = 1000 // 4


def _rsqrt_nr(x):
    xh = x * jnp.float32(0.5)
    i = plsc.bitcast(x, jnp.int32)
    i = jnp.int32(0x5F3759DF) - jnp.right_shift(i, jnp.int32(1))
    y = plsc.bitcast(i, jnp.float32)
    for _ in range(3):
        y = y * (jnp.float32(1.5) - xh * y * y)
    return y


def _transe_body(head, tail, rel, n_head, n_tail, n_rel, ent, rel_emb, out,
                 ix0, ix1, ix2, ix3, ix4, ix5,
                 gx0, gx1, gx2, gx3, gx4, gx5,
                 gr0, gr1, gr2, gr3, gr4, gr5,
                 outv, sem):
    wid = lax.axis_index("s") * _NC + lax.axis_index("c")
    base = wid * _C

    srcs = (head, tail, rel, n_head, n_tail, n_rel)
    tables = (ent, ent, rel_emb, ent, ent, rel_emb)
    ix = (ix0, ix1, ix2, ix3, ix4, ix5)
    gx = (gx0, gx1, gx2, gx3, gx4, gx5)
    gr = (gr0, gr1, gr2, gr3, gr4, gr5)

    zero = jnp.zeros((_CHUNK,), jnp.float32)
    iota = lax.iota(jnp.int32, _CHUNK)

    def group_body(j, carry):
        gbase = base + j * _G
        for t in range(6):
            pltpu.sync_copy(srcs[t].at[pl.ds(gbase, _G)], ix[t])
        for t in range(6):
            for k in range(_NK):
                sl = pl.ds(k * _CHUNK, _CHUNK)
                gx[t][sl] = jnp.right_shift(ix[t][sl], jnp.int32(2))
        handles = [
            pltpu.async_copy(tables[t].at[gx[t]], gr[t], sem)
            for t in range(6)
        ]
        for h in handles:
            h.wait()

        def chunk_body(k, carry3):
            rows = k * _CHUNK + iota
            sl = pl.ds(k * _CHUNK, _CHUNK)
            cb = [
                jnp.bitwise_and(ix[t][sl], jnp.int32(3)) * jnp.int32(_D)
                for t in range(6)
            ]
            hh = tt = rr = hr = ht = rt = zero
            mhh = mtt = mrr = mhr = mht = mrt = zero
            for s in range(_D):
                diag = jnp.bitwise_and(iota + jnp.int32(s), jnp.int32(_D - 1))
                h = plsc.load_gather(gr0, [rows, cb[0] + diag])
                t_ = plsc.load_gather(gr1, [rows, cb[1] + diag])
                r_ = plsc.load_gather(gr2, [rows, cb[2] + diag])
                nh = plsc.load_gather(gr3, [rows, cb[3] + diag])
                nt = plsc.load_gather(gr4, [rows, cb[4] + diag])
                nr = plsc.load_gather(gr5, [rows, cb[5] + diag])
                hh = hh + h * h
                tt = tt + t_ * t_
                rr = rr + r_ * r_
                hr = hr + h * r_
                ht = ht + h * t_
                rt = rt + r_ * t_
                mhh = mhh + nh * nh
                mtt = mtt + nt * nt
                mrr = mrr + nr * nr
                mhr = mhr + nh * nr
                mht = mht + nh * nt
                mrt = mrt + nr * nt
            two = jnp.float32(2.0)
            three = jnp.float32(3.0)
            eps = jnp.float32(1e-30)
            pos2 = three + two * (hr * _rsqrt_nr(hh * rr)
                                  - ht * _rsqrt_nr(hh * tt)
                                  - rt * _rsqrt_nr(rr * tt))
            neg2 = three + two * (mhr * _rsqrt_nr(mhh * mrr)
                                  - mht * _rsqrt_nr(mhh * mtt)
                                  - mrt * _rsqrt_nr(mrr * mtt))
            pos2 = jnp.maximum(pos2, eps)
            neg2 = jnp.maximum(neg2, eps)
            pos = pos2 * _rsqrt_nr(pos2)
            neg = neg2 * _rsqrt_nr(neg2)
            outv[pl.ds((j * _NK + k) * _CHUNK, _CHUNK)] = (
                jnp.float32(_GAMMA) + pos - neg)
            return carry3

        lax.fori_loop(0, _NK, chunk_body, 0)
        return carry

    lax.fori_loop(0, _NG, group_body, 0)
    pltpu.sync_copy(outv, out.at[pl.ds(base, _C)])


_transe_sc = functools.partial(
    pl.kernel,
    mesh=plsc.VectorSubcoreMesh(core_axis_name="c", subcore_axis_name="s"),
    compiler_params=pltpu.CompilerParams(needs_layout_passes=False),
    out_type=jax.ShapeDtypeStruct((_B,), jnp.float32),
    scratch_types=(
        [pltpu.VMEM((_G,), jnp.int32) for _ in range(6)]
        + [pltpu.VMEM((_G,), jnp.int32) for _ in range(6)]
        + [pltpu.VMEM((_G, 128), jnp.float32) for _ in range(6)]
        + [pltpu.VMEM((_C,), jnp.float32),
           pltpu.SemaphoreType.DMA]
    ),
)(_transe_body)


def kernel(head, tail, relation, n_head, n_tail, n_relation, entity_embed, relation_embed):
    return _transe_sc(
        head.astype(jnp.int32),
        tail.astype(jnp.int32),
        relation.astype(jnp.int32),
        n_head.astype(jnp.int32),
        n_tail.astype(jnp.int32),
        n_relation.astype(jnp.int32),
        entity_embed.reshape(_E4, 128),
        relation_embed.reshape(_R4, 128),
    )


# speedup vs baseline: 4.8358x; 1.0296x over previous
import functools

import jax
import jax.numpy as jnp
from jax import lax
from jax.experimental import pallas as pl
from jax.experimental.pallas import tpu as pltpu
from jax.experimental.pallas import tpu_sc as plsc

_B = 16384
_D = 32
_GAMMA = 1.0
_NC = 2
_NS = 16
_NW = _NC * _NS
_C = _B // _NW
_G = 64
_NG = _C // _G
_CHUNK = 16
_NK = _G // _CHUNK
_E4 = 1000000 // 4
_R4 = 1000 // 4


def _rsqrt_nr(x):
    xh = x * jnp.float32(0.5)
    i = plsc.bitcast(x, jnp.int32)
    i = jnp.int32(0x5F3759DF) - jnp.right_shift(i, jnp.int32(1))
    y = plsc.bitcast(i, jnp.float32)
    for _ in range(3):
        y = y * (jnp.float32(1.5) - xh * y * y)
    return y


def _transe_body(head, tail, rel, n_head, n_tail, n_rel, ent, rel_emb, out,
                 *refs):
    wid = lax.axis_index("s") * _NC + lax.axis_index("c")
    base = wid * _C

    srcs = (head, tail, rel, n_head, n_tail, n_rel)
    tables = (ent, ent, rel_emb, ent, ent, rel_emb)
    ix = (refs[0:6], refs[6:12])
    gx = (refs[12:18], refs[18:24])
    gr = (refs[24:30], refs[30:36])
    outv = refs[36]
    sem = refs[37]

    zero = jnp.zeros((_CHUNK,), jnp.float32)
    iota = lax.iota(jnp.int32, _CHUNK)

    def stage_group(j, par):
        gbase = base + j * _G
        for t in range(6):
            pltpu.sync_copy(srcs[t].at[pl.ds(gbase, _G)], ix[par][t])
        for t in range(6):
            for k in range(_NK):
                sl = pl.ds(k * _CHUNK, _CHUNK)
                gx[par][t][sl] = jnp.right_shift(
                    ix[par][t][sl], jnp.int32(2))
        return [
            pltpu.async_copy(tables[t].at[gx[par][t]], gr[par][t], sem)
            for t in range(6)
        ]

    def group_compute(j, par):
        grp = gr[par]

        def chunk_body(k, carry3):
            rows = k * _CHUNK + iota
            sl = pl.ds(k * _CHUNK, _CHUNK)
            cb = [
                jnp.bitwise_and(ix[par][t][sl], jnp.int32(3))
                * jnp.int32(_D)
                for t in range(6)
            ]
            hh = tt = rr = hr = ht = rt = zero
            mhh = mtt = mrr = mhr = mht = mrt = zero
            for s in range(_D):
                diag = jnp.bitwise_and(iota + jnp.int32(s), jnp.int32(_D - 1))
                h = plsc.load_gather(grp[0], [rows, cb[0] + diag])
                t_ = plsc.load_gather(grp[1], [rows, cb[1] + diag])
                r_ = plsc.load_gather(grp[2], [rows, cb[2] + diag])
                nh = plsc.load_gather(grp[3], [rows, cb[3] + diag])
                nt = plsc.load_gather(grp[4], [rows, cb[4] + diag])
                nr = plsc.load_gather(grp[5], [rows, cb[5] + diag])
                hh = hh + h * h
                tt = tt + t_ * t_
                rr = rr + r_ * r_
                hr = hr + h * r_
                ht = ht + h * t_
                rt = rt + r_ * t_
                mhh = mhh + nh * nh
                mtt = mtt + nt * nt
                mrr = mrr + nr * nr
                mhr = mhr + nh * nr
                mht = mht + nh * nt
                mrt = mrt + nr * nt
            two = jnp.float32(2.0)
            three = jnp.float32(3.0)
            eps = jnp.float32(1e-30)
            pos2 = three + two * (hr * _rsqrt_nr(hh * rr)
                                  - ht * _rsqrt_nr(hh * tt)
                                  - rt * _rsqrt_nr(rr * tt))
            neg2 = three + two * (mhr * _rsqrt_nr(mhh * mrr)
                                  - mht * _rsqrt_nr(mhh * mtt)
                                  - mrt * _rsqrt_nr(mrr * mtt))
            pos2 = jnp.maximum(pos2, eps)
            neg2 = jnp.maximum(neg2, eps)
            pos = pos2 * _rsqrt_nr(pos2)
            neg = neg2 * _rsqrt_nr(neg2)
            outv[pl.ds((j * _NK + k) * _CHUNK, _CHUNK)] = (
                jnp.float32(_GAMMA) + pos - neg)
            return carry3

        lax.fori_loop(0, _NK, chunk_body, 0)

    handles = stage_group(0, 0)
    for j in range(_NG):
        nxt = None
        if j + 1 < _NG:
            nxt = stage_group(j + 1, (j + 1) % 2)
        for h in handles:
            h.wait()
        group_compute(j, j % 2)
        handles = nxt

    pltpu.sync_copy(outv, out.at[pl.ds(base, _C)])


_transe_sc = functools.partial(
    pl.kernel,
    mesh=plsc.VectorSubcoreMesh(core_axis_name="c", subcore_axis_name="s"),
    compiler_params=pltpu.CompilerParams(needs_layout_passes=False),
    out_type=jax.ShapeDtypeStruct((_B,), jnp.float32),
    scratch_types=(
        [pltpu.VMEM((_G,), jnp.int32) for _ in range(12)]
        + [pltpu.VMEM((_G,), jnp.int32) for _ in range(12)]
        + [pltpu.VMEM((_G, 128), jnp.float32) for _ in range(12)]
        + [pltpu.VMEM((_C,), jnp.float32),
           pltpu.SemaphoreType.DMA]
    ),
)(_transe_body)


def kernel(head, tail, relation, n_head, n_tail, n_relation, entity_embed, relation_embed):
    return _transe_sc(
        head.astype(jnp.int32),
        tail.astype(jnp.int32),
        relation.astype(jnp.int32),
        n_head.astype(jnp.int32),
        n_tail.astype(jnp.int32),
        n_relation.astype(jnp.int32),
        entity_embed.reshape(_E4, 128),
        relation_embed.reshape(_R4, 128),
    )
